# baseline (speedup 1.0000x reference)
import jax
import jax.numpy as jnp
from jax import lax
from jax.experimental import pallas as pl

NUM_BITS = 8
MIN_VAL = 0.0
MAX_VAL = 1.0
THRESHOLD = 3.0
B = 1024
F = 512
HIDDEN = 2048
CLASSES = 1000
ENC = F * NUM_BITS

BB = 128


def _body(x_ref, w0_ref, w1_ref, oc_ref, out_ref, pred_ref):
    xb = x_ref[...]
    xc = jnp.clip(xb, MIN_VAL, MAX_VAL)
    norm = (xc - MIN_VAL) / (MAX_VAL - MIN_VAL)
    lv = jnp.round(norm * (2 ** NUM_BITS - 1)).astype(jnp.int32)
    gray = lv ^ (lv >> 1)
    planes = [((gray >> k) & 1).astype(jnp.bfloat16) for k in range(NUM_BITS)]
    enc = jnp.concatenate(planes, axis=1)

    z0 = lax.dot_general(enc, w0_ref[...], (((1,), (1,)), ((), ())),
                         preferred_element_type=jnp.float32)
    a0 = (z0 >= THRESHOLD).astype(jnp.bfloat16)
    z1 = lax.dot_general(a0, w1_ref[...], (((1,), (1,)), ((), ())),
                         preferred_element_type=jnp.float32)
    a1 = (z1 >= THRESHOLD).astype(jnp.float32)

    a01 = jnp.concatenate([a0.astype(jnp.float32), a1], axis=1)
    out = lax.dot_general(a01, oc_ref[...], (((1,), (0,)), ((), ())),
                          preferred_element_type=jnp.float32)
    out_ref[...] = out

    mx = jnp.max(out, axis=1, keepdims=True)
    idx = lax.broadcasted_iota(jnp.int32, out.shape, 1)
    pred = jnp.min(jnp.where(out == mx, idx, CLASSES), axis=1)
    pred_ref[...] = pred.reshape(1, 1, BB).astype(jnp.int32)


def kernel(trainOrTest, x, y, W0, W1, outC):
    del trainOrTest, y
    w0b = (W0.reshape(HIDDEN, F, NUM_BITS)
             .swapaxes(1, 2)
             .reshape(HIDDEN, ENC)
             .astype(jnp.bfloat16))
    w1b = W1.astype(jnp.bfloat16)
    ocr = outC.reshape(2 * HIDDEN, CLASSES)

    grid = (B // BB,)
    out_act, preds3 = pl.pallas_call(
        _body,
        grid=grid,
        in_specs=[
            pl.BlockSpec((BB, F), lambda i: (i, 0)),
            pl.BlockSpec((HIDDEN, ENC), lambda i: (0, 0)),
            pl.BlockSpec((HIDDEN, HIDDEN), lambda i: (0, 0)),
            pl.BlockSpec((2 * HIDDEN, CLASSES), lambda i: (0, 0)),
        ],
        out_specs=[
            pl.BlockSpec((BB, CLASSES), lambda i: (i, 0)),
            pl.BlockSpec((1, 1, BB), lambda i: (i, 0, 0)),
        ],
        out_shape=[
            jax.ShapeDtypeStruct((B, CLASSES), jnp.float32),
            jax.ShapeDtypeStruct((B // BB, 1, BB), jnp.int32),
        ],
    )(x, w0b, w1b, ocr)

    predictions = preds3.reshape(B)
    return predictions, out_act


# speedup vs baseline: 1.9263x; 1.9263x over previous
import jax
import jax.numpy as jnp
from jax import lax
from jax.experimental import pallas as pl
from jax.experimental.pallas import tpu as pltpu

NUM_BITS = 8
MIN_VAL = 0.0
MAX_VAL = 1.0
THRESHOLD = 3.0
B = 1024
F = 512
HIDDEN = 2048
CLASSES = 1000
ENC = F * NUM_BITS

BB = 128
HB = 256


def _encode_body(x_ref, enc_ref, r_ref):
    @pl.when(pl.program_id(0) == 0)
    def _():
        src = lax.broadcasted_iota(jnp.int32, (F, ENC), 1) // NUM_BITS
        dst = lax.broadcasted_iota(jnp.int32, (F, ENC), 0)
        r_ref[...] = (src == dst).astype(jnp.bfloat16)

    xb = x_ref[...]
    xc = jnp.clip(xb, MIN_VAL, MAX_VAL)
    norm = (xc - MIN_VAL) / (MAX_VAL - MIN_VAL)
    lv = jnp.round(norm * (2 ** NUM_BITS - 1)).astype(jnp.int32)
    gray = lv ^ (lv >> 1)
    gray_rep = lax.dot_general(gray.astype(jnp.bfloat16), r_ref[...],
                               (((1,), (0,)), ((), ())),
                               preferred_element_type=jnp.float32)
    gi = gray_rep.astype(jnp.int32)
    kidx = lax.broadcasted_iota(jnp.int32, (BB, ENC), 1) & (NUM_BITS - 1)
    enc_ref[...] = ((gi >> kidx) & 1).astype(jnp.bfloat16)


def _layer_body(act_ref, w_ref, out_ref):
    wb = w_ref[...].astype(jnp.bfloat16)
    z = lax.dot_general(act_ref[...], wb, (((1,), (1,)), ((), ())),
                        preferred_element_type=jnp.float32)
    out_ref[...] = (z >= THRESHOLD).astype(jnp.bfloat16)


def _out_body(a0_ref, a1_ref, oc_ref, out_ref, pred_ref):
    a01 = jnp.concatenate(
        [a0_ref[...].astype(jnp.float32), a1_ref[...].astype(jnp.float32)],
        axis=1)
    out = lax.dot_general(a01, oc_ref[...], (((1,), (0,)), ((), ())),
                          preferred_element_type=jnp.float32)
    out_ref[...] = out
    mx = jnp.max(out, axis=1, keepdims=True)
    idx = lax.broadcasted_iota(jnp.int32, out.shape, 1)
    pred = jnp.min(jnp.where(out == mx, idx, CLASSES), axis=1)
    pred_ref[...] = pred.reshape(1, 1, BB).astype(jnp.int32)


def kernel(trainOrTest, x, y, W0, W1, outC):
    del trainOrTest, y
    ocr = outC.reshape(2 * HIDDEN, CLASSES)

    enc = pl.pallas_call(
        _encode_body,
        grid=(B // BB,),
        in_specs=[pl.BlockSpec((BB, F), lambda i: (i, 0))],
        out_specs=pl.BlockSpec((BB, ENC), lambda i: (i, 0)),
        out_shape=jax.ShapeDtypeStruct((B, ENC), jnp.bfloat16),
        scratch_shapes=[pltpu.VMEM((F, ENC), jnp.bfloat16)],
    )(x)

    a0 = pl.pallas_call(
        _layer_body,
        grid=(HIDDEN // HB,),
        in_specs=[
            pl.BlockSpec((B, ENC), lambda h: (0, 0)),
            pl.BlockSpec((HB, ENC), lambda h: (h, 0)),
        ],
        out_specs=pl.BlockSpec((B, HB), lambda h: (0, h)),
        out_shape=jax.ShapeDtypeStruct((B, HIDDEN), jnp.bfloat16),
    )(enc, W0)

    a1 = pl.pallas_call(
        _layer_body,
        grid=(HIDDEN // HB,),
        in_specs=[
            pl.BlockSpec((B, HIDDEN), lambda h: (0, 0)),
            pl.BlockSpec((HB, HIDDEN), lambda h: (h, 0)),
        ],
        out_specs=pl.BlockSpec((B, HB), lambda h: (0, h)),
        out_shape=jax.ShapeDtypeStruct((B, HIDDEN), jnp.bfloat16),
    )(a0, W1)

    out_act, preds3 = pl.pallas_call(
        _out_body,
        grid=(B // BB,),
        in_specs=[
            pl.BlockSpec((BB, HIDDEN), lambda i: (i, 0)),
            pl.BlockSpec((BB, HIDDEN), lambda i: (i, 0)),
            pl.BlockSpec((2 * HIDDEN, CLASSES), lambda i: (0, 0)),
        ],
        out_specs=[
            pl.BlockSpec((BB, CLASSES), lambda i: (i, 0)),
            pl.BlockSpec((1, 1, BB), lambda i: (i, 0, 0)),
        ],
        out_shape=[
            jax.ShapeDtypeStruct((B, CLASSES), jnp.float32),
            jax.ShapeDtypeStruct((B // BB, 1, BB), jnp.int32),
        ],
    )(a0, a1, ocr)

    predictions = preds3.reshape(B)
    return predictions, out_act


# speedup vs baseline: 2.0240x; 1.0507x over previous
import jax
import jax.numpy as jnp
from jax import lax
from jax.experimental import pallas as pl
from jax.experimental.pallas import tpu as pltpu

NUM_BITS = 8
MIN_VAL = 0.0
MAX_VAL = 1.0
THRESHOLD = 3.0
B = 1024
F = 512
HIDDEN = 2048
CLASSES = 1000
ENC = F * NUM_BITS

BB = 128
HB = 256


def _encode_body(x_ref, enc_ref, r_ref):
    @pl.when(pl.program_id(0) == 0)
    def _():
        src = lax.broadcasted_iota(jnp.int32, (F, ENC), 1) // NUM_BITS
        dst = lax.broadcasted_iota(jnp.int32, (F, ENC), 0)
        r_ref[...] = (src == dst).astype(jnp.bfloat16)

    xb = x_ref[...]
    xc = jnp.clip(xb, MIN_VAL, MAX_VAL)
    norm = (xc - MIN_VAL) / (MAX_VAL - MIN_VAL)
    lv = jnp.round(norm * (2 ** NUM_BITS - 1)).astype(jnp.int32)
    gray = lv ^ (lv >> 1)
    gray_rep = lax.dot_general(gray.astype(jnp.bfloat16), r_ref[...],
                               (((1,), (0,)), ((), ())),
                               preferred_element_type=jnp.float32)
    gi = gray_rep.astype(jnp.int32)
    kidx = lax.broadcasted_iota(jnp.int32, (BB, ENC), 1) & (NUM_BITS - 1)
    enc_ref[...] = ((gi >> kidx) & 1).astype(jnp.bfloat16)


def _layer_body(act_ref, w_ref, out_ref):
    wb = w_ref[...].astype(jnp.bfloat16)
    z = lax.dot_general(act_ref[...], wb, (((1,), (1,)), ((), ())),
                        preferred_element_type=jnp.float32)
    out_ref[...] = (z >= THRESHOLD).astype(jnp.bfloat16)


def _out_body(a0_ref, a1_ref, oc_ref, out_ref, pred_ref):
    out = lax.dot_general(a0_ref[...].astype(jnp.float32), oc_ref[0],
                          (((1,), (0,)), ((), ())),
                          preferred_element_type=jnp.float32)
    out = out + lax.dot_general(a1_ref[...].astype(jnp.float32), oc_ref[1],
                                (((1,), (0,)), ((), ())),
                                preferred_element_type=jnp.float32)
    out_ref[...] = out
    mx = jnp.max(out, axis=1, keepdims=True)
    idx = lax.broadcasted_iota(jnp.int32, out.shape, 1)
    pred = jnp.min(jnp.where(out == mx, idx, CLASSES), axis=1)
    pred_ref[...] = pred.reshape(1, 1, BB).astype(jnp.int32)


def kernel(trainOrTest, x, y, W0, W1, outC):
    del trainOrTest, y

    enc = pl.pallas_call(
        _encode_body,
        grid=(B // BB,),
        in_specs=[pl.BlockSpec((BB, F), lambda i: (i, 0))],
        out_specs=pl.BlockSpec((BB, ENC), lambda i: (i, 0)),
        out_shape=jax.ShapeDtypeStruct((B, ENC), jnp.bfloat16),
        scratch_shapes=[pltpu.VMEM((F, ENC), jnp.bfloat16)],
    )(x)

    a0 = pl.pallas_call(
        _layer_body,
        grid=(HIDDEN // HB,),
        in_specs=[
            pl.BlockSpec((B, ENC), lambda h: (0, 0)),
            pl.BlockSpec((HB, ENC), lambda h: (h, 0)),
        ],
        out_specs=pl.BlockSpec((B, HB), lambda h: (0, h)),
        out_shape=jax.ShapeDtypeStruct((B, HIDDEN), jnp.bfloat16),
    )(enc, W0)

    a1 = pl.pallas_call(
        _layer_body,
        grid=(HIDDEN // HB,),
        in_specs=[
            pl.BlockSpec((B, HIDDEN), lambda h: (0, 0)),
            pl.BlockSpec((HB, HIDDEN), lambda h: (h, 0)),
        ],
        out_specs=pl.BlockSpec((B, HB), lambda h: (0, h)),
        out_shape=jax.ShapeDtypeStruct((B, HIDDEN), jnp.bfloat16),
    )(a0, W1)

    out_act, preds3 = pl.pallas_call(
        _out_body,
        grid=(B // BB,),
        in_specs=[
            pl.BlockSpec((BB, HIDDEN), lambda i: (i, 0)),
            pl.BlockSpec((BB, HIDDEN), lambda i: (i, 0)),
            pl.BlockSpec((2, HIDDEN, CLASSES), lambda i: (0, 0, 0)),
        ],
        out_specs=[
            pl.BlockSpec((BB, CLASSES), lambda i: (i, 0)),
            pl.BlockSpec((1, 1, BB), lambda i: (i, 0, 0)),
        ],
        out_shape=[
            jax.ShapeDtypeStruct((B, CLASSES), jnp.float32),
            jax.ShapeDtypeStruct((B // BB, 1, BB), jnp.int32),
        ],
    )(a0, a1, outC)

    predictions = preds3.reshape(B)
    return predictions, out_act


# speedup vs baseline: 2.1750x; 1.0746x over previous
import jax
import jax.numpy as jnp
from jax import lax
from jax.experimental import pallas as pl
from jax.experimental.pallas import tpu as pltpu

NUM_BITS = 8
MIN_VAL = 0.0
MAX_VAL = 1.0
THRESHOLD = 3.0
B = 1024
F = 512
HIDDEN = 2048
CLASSES = 1000
ENC = F * NUM_BITS

BB = 128
HB = 256

NB = B // BB
NH = HIDDEN // HB
S_Z0 = NB
S_Z1 = S_Z0 + NH
S_OUT = S_Z1 + NH
S_ARGMAX = S_OUT + 2 * NH
N_STEPS = S_ARGMAX + 1


def _body(x_ref, w0_ref, w1_ref, oc_ref, out_ref, pred_ref,
          enc_s, a0_s, a1_s, r_s):
    i = pl.program_id(0)

    @pl.when(i == 0)
    def _():
        src = lax.broadcasted_iota(jnp.int32, (F, ENC), 1) // NUM_BITS
        dst = lax.broadcasted_iota(jnp.int32, (F, ENC), 0)
        r_s[...] = (src == dst).astype(jnp.bfloat16)

    @pl.when(i < S_Z0)
    def _():
        xb = x_ref[...]
        xc = jnp.clip(xb, MIN_VAL, MAX_VAL)
        norm = (xc - MIN_VAL) / (MAX_VAL - MIN_VAL)
        lv = jnp.round(norm * (2 ** NUM_BITS - 1)).astype(jnp.int32)
        gray = lv ^ (lv >> 1)
        rep = lax.dot_general(gray.astype(jnp.bfloat16), r_s[...],
                              (((1,), (0,)), ((), ())),
                              preferred_element_type=jnp.float32)
        gi = rep.astype(jnp.int32)
        kidx = lax.broadcasted_iota(jnp.int32, (BB, ENC), 1) & (NUM_BITS - 1)
        enc_s[pl.ds(i * BB, BB), :] = ((gi >> kidx) & 1).astype(jnp.bfloat16)

    @pl.when((i >= S_Z0) & (i < S_Z1))
    def _():
        h = i - S_Z0
        wb = w0_ref[...].astype(jnp.bfloat16)
        z = lax.dot_general(enc_s[...], wb, (((1,), (1,)), ((), ())),
                            preferred_element_type=jnp.float32)
        a0_s[:, pl.ds(h * HB, HB)] = (z >= THRESHOLD).astype(jnp.bfloat16)

    @pl.when((i >= S_Z1) & (i < S_OUT))
    def _():
        h = i - S_Z1
        wb = w1_ref[...].astype(jnp.bfloat16)
        z = lax.dot_general(a0_s[...], wb, (((1,), (1,)), ((), ())),
                            preferred_element_type=jnp.float32)
        a1_s[:, pl.ds(h * HB, HB)] = (z >= THRESHOLD).astype(jnp.bfloat16)

    @pl.when((i >= S_OUT) & (i < S_ARGMAX))
    def _():
        j = i - S_OUT
        h = jnp.where(j < NH, j, j - NH)
        ocb = oc_ref[0]

        def acc(a_s):
            ab = a_s[:, pl.ds(h * HB, HB)].astype(jnp.float32)
            return lax.dot_general(ab, ocb, (((1,), (0,)), ((), ())),
                                   preferred_element_type=jnp.float32)

        @pl.when(j < NH)
        def _():
            p = acc(a0_s)

            @pl.when(j == 0)
            def _():
                out_ref[...] = p

            @pl.when(j > 0)
            def _():
                out_ref[...] = out_ref[...] + p

        @pl.when(j >= NH)
        def _():
            out_ref[...] = out_ref[...] + acc(a1_s)

    @pl.when(i == S_ARGMAX)
    def _():
        out = out_ref[...]
        mx = jnp.max(out, axis=1, keepdims=True)
        idx = lax.broadcasted_iota(jnp.int32, out.shape, 1)
        pred = jnp.min(jnp.where(out == mx, idx, CLASSES), axis=1)
        pred_ref[...] = pred.reshape(NB, 1, BB).astype(jnp.int32)


def kernel(trainOrTest, x, y, W0, W1, outC):
    del trainOrTest, y

    def oc_index(i):
        j = jnp.clip(i - S_OUT, 0, 2 * NH - 1)
        return (j // NH, jnp.where(j < NH, j, j - NH), 0)

    out_act, preds3 = pl.pallas_call(
        _body,
        grid=(N_STEPS,),
        in_specs=[
            pl.BlockSpec((BB, F), lambda i: (jnp.minimum(i, NB - 1), 0)),
            pl.BlockSpec((HB, ENC),
                         lambda i: (jnp.clip(i - S_Z0, 0, NH - 1), 0)),
            pl.BlockSpec((HB, HIDDEN),
                         lambda i: (jnp.clip(i - S_Z1, 0, NH - 1), 0)),
            pl.BlockSpec((1, HB, CLASSES), oc_index),
        ],
        out_specs=[
            pl.BlockSpec((B, CLASSES), lambda i: (0, 0)),
            pl.BlockSpec((NB, 1, BB), lambda i: (0, 0, 0)),
        ],
        out_shape=[
            jax.ShapeDtypeStruct((B, CLASSES), jnp.float32),
            jax.ShapeDtypeStruct((NB, 1, BB), jnp.int32),
        ],
        scratch_shapes=[
            pltpu.VMEM((B, ENC), jnp.bfloat16),
            pltpu.VMEM((B, HIDDEN), jnp.bfloat16),
            pltpu.VMEM((B, HIDDEN), jnp.bfloat16),
            pltpu.VMEM((F, ENC), jnp.bfloat16),
        ],
    )(x, W0, W1, outC)

    predictions = preds3.reshape(B)
    return predictions, out_act


# speedup vs baseline: 2.3202x; 1.0668x over previous
import jax
import jax.numpy as jnp
from jax import lax
from jax.experimental import pallas as pl
from jax.experimental.pallas import tpu as pltpu

NUM_BITS = 8
MIN_VAL = 0.0
MAX_VAL = 1.0
THRESHOLD = 3.0
B = 1024
F = 512
HIDDEN = 2048
CLASSES = 1000
ENC = F * NUM_BITS

BB = 128
HB = 256

NB = B // BB
NH = HIDDEN // HB
S_Z0 = NB
S_Z1 = S_Z0 + NH // 2
S_OUT = S_Z1 + NH // 2
S_ARGMAX = S_OUT + NH
N_STEPS = S_ARGMAX + 1


def _body(x_ref, w0a_ref, w0b_ref, w1a_ref, w1b_ref, oca_ref, ocb_ref,
          out_ref, pred_ref, enc_s, a0_s, a1_s, r_s):
    i = pl.program_id(0)

    @pl.when(i == 0)
    def _():
        src = lax.broadcasted_iota(jnp.int32, (F, ENC), 1) // NUM_BITS
        dst = lax.broadcasted_iota(jnp.int32, (F, ENC), 0)
        r_s[...] = (src == dst).astype(jnp.bfloat16)

    @pl.when(i < S_Z0)
    def _():
        xb = x_ref[...]
        xc = jnp.clip(xb, MIN_VAL, MAX_VAL)
        norm = (xc - MIN_VAL) / (MAX_VAL - MIN_VAL)
        lv = jnp.round(norm * (2 ** NUM_BITS - 1)).astype(jnp.int32)
        gray = lv ^ (lv >> 1)
        rep = lax.dot_general(gray.astype(jnp.bfloat16), r_s[...],
                              (((1,), (0,)), ((), ())),
                              preferred_element_type=jnp.float32)
        gi = rep.astype(jnp.int32)
        kidx = lax.broadcasted_iota(jnp.int32, (BB, ENC), 1) & (NUM_BITS - 1)
        enc_s[pl.ds(i * BB, BB), :] = ((gi >> kidx) & 1).astype(jnp.bfloat16)

    def layer_step(step0, act_s, wa_ref, wb_ref, dst_s):
        h2 = (i - step0) * 2
        for h, wref in ((h2, wa_ref), (h2 + 1, wb_ref)):
            wb = wref[...].astype(jnp.bfloat16)
            z = lax.dot_general(act_s[...], wb, (((1,), (1,)), ((), ())),
                                preferred_element_type=jnp.float32)
            dst_s[:, pl.ds(h * HB, HB)] = (z >= THRESHOLD).astype(jnp.bfloat16)

    @pl.when((i >= S_Z0) & (i < S_Z1))
    def _():
        layer_step(S_Z0, enc_s, w0a_ref, w0b_ref, a0_s)

    @pl.when((i >= S_Z1) & (i < S_OUT))
    def _():
        layer_step(S_Z1, a0_s, w1a_ref, w1b_ref, a1_s)

    @pl.when((i >= S_OUT) & (i < S_ARGMAX))
    def _():
        j = i - S_OUT
        h2 = jnp.where(j < NH // 2, j, j - NH // 2) * 2

        def acc(a_s):
            p = lax.dot_general(
                a_s[:, pl.ds(h2 * HB, HB)].astype(jnp.float32), oca_ref[0],
                (((1,), (0,)), ((), ())), preferred_element_type=jnp.float32)
            return p + lax.dot_general(
                a_s[:, pl.ds((h2 + 1) * HB, HB)].astype(jnp.float32),
                ocb_ref[0],
                (((1,), (0,)), ((), ())), preferred_element_type=jnp.float32)

        @pl.when(j < NH // 2)
        def _():
            p = acc(a0_s)

            @pl.when(j == 0)
            def _():
                out_ref[...] = p

            @pl.when(j > 0)
            def _():
                out_ref[...] = out_ref[...] + p

        @pl.when(j >= NH // 2)
        def _():
            out_ref[...] = out_ref[...] + acc(a1_s)

    @pl.when(i == S_ARGMAX)
    def _():
        out = out_ref[...]
        mx = jnp.max(out, axis=1, keepdims=True)
        idx = lax.broadcasted_iota(jnp.int32, out.shape, 1)
        pred = jnp.min(jnp.where(out == mx, idx, CLASSES), axis=1)
        pred_ref[...] = pred.reshape(NB, 1, BB).astype(jnp.int32)


def kernel(trainOrTest, x, y, W0, W1, outC):
    del trainOrTest, y

    def w_index(step0, off):
        def f(i):
            return (jnp.clip(i - step0, 0, NH // 2 - 1) * 2 + off, 0)
        return f

    def oc_index(off):
        def f(i):
            j = jnp.clip(i - S_OUT, 0, NH - 1)
            layer = j // (NH // 2)
            h2 = jnp.where(j < NH // 2, j, j - NH // 2) * 2
            return (layer, h2 + off, 0)
        return f

    out_act, preds3 = pl.pallas_call(
        _body,
        grid=(N_STEPS,),
        in_specs=[
            pl.BlockSpec((BB, F), lambda i: (jnp.minimum(i, NB - 1), 0)),
            pl.BlockSpec((HB, ENC), w_index(S_Z0, 0)),
            pl.BlockSpec((HB, ENC), w_index(S_Z0, 1)),
            pl.BlockSpec((HB, HIDDEN), w_index(S_Z1, 0)),
            pl.BlockSpec((HB, HIDDEN), w_index(S_Z1, 1)),
            pl.BlockSpec((1, HB, CLASSES), oc_index(0)),
            pl.BlockSpec((1, HB, CLASSES), oc_index(1)),
        ],
        out_specs=[
            pl.BlockSpec((B, CLASSES), lambda i: (0, 0)),
            pl.BlockSpec((NB, 1, BB), lambda i: (0, 0, 0)),
        ],
        out_shape=[
            jax.ShapeDtypeStruct((B, CLASSES), jnp.float32),
            jax.ShapeDtypeStruct((NB, 1, BB), jnp.int32),
        ],
        scratch_shapes=[
            pltpu.VMEM((B, ENC), jnp.bfloat16),
            pltpu.VMEM((B, HIDDEN), jnp.bfloat16),
            pltpu.VMEM((B, HIDDEN), jnp.bfloat16),
            pltpu.VMEM((F, ENC), jnp.bfloat16),
        ],
    )(x, W0, W0, W1, W1, outC, outC)

    predictions = preds3.reshape(B)
    return predictions, out_act
